# Initial kernel scaffold; baseline (speedup 1.0000x reference)
import jax
import jax.numpy as jnp
from jax.experimental import pallas as pl


def kernel(x, fc1_weight, fc2_weight, indices, counts):
    raise NotImplementedError("write your pallas kernel here")


# speedup vs baseline: 4.1524x; 4.1524x over previous
import functools

import jax
import jax.numpy as jnp
from jax.experimental import pallas as pl
from jax.experimental.pallas import tpu as pltpu

E = 8
TOP_K = 2
D_MODEL = 1024
D_FF = 2048
N_TOK = 2048
N = N_TOK * TOP_K
TILE_M = 256
TMAX = (E - 1) + (N + TILE_M - 1) // TILE_M
ZROWS = TMAX * TILE_M


def _ffn_tile_kernel(emap_ref, z_ref, w1_ref, wg_ref, w2_ref, o_ref):
    del emap_ref
    zt = z_ref[...]
    dn = (((1,), (1,)), ((), ()))
    h1 = jax.lax.dot_general(zt, w1_ref[0], dn,
                             preferred_element_type=jnp.float32)
    gate = jax.lax.dot_general(zt, wg_ref[0], dn,
                               preferred_element_type=jnp.float32)
    act = h1 * (gate * jax.nn.sigmoid(gate))
    o_ref[...] = jax.lax.dot_general(act, w2_ref[0], dn,
                                     preferred_element_type=jnp.float32)


def _grouped_ffn(z, w1, wg, w2, emap):
    grid_spec = pltpu.PrefetchScalarGridSpec(
        num_scalar_prefetch=1,
        grid=(TMAX,),
        in_specs=[
            pl.BlockSpec((TILE_M, D_MODEL), lambda t, m: (t, 0)),
            pl.BlockSpec((1, D_FF, D_MODEL), lambda t, m: (m[t], 0, 0)),
            pl.BlockSpec((1, D_FF, D_MODEL), lambda t, m: (m[t], 0, 0)),
            pl.BlockSpec((1, D_MODEL, D_FF), lambda t, m: (m[t], 0, 0)),
        ],
        out_specs=pl.BlockSpec((TILE_M, D_MODEL), lambda t, m: (t, 0)),
    )
    return pl.pallas_call(
        _ffn_tile_kernel,
        grid_spec=grid_spec,
        out_shape=jax.ShapeDtypeStruct((ZROWS, D_MODEL), jnp.float32),
        compiler_params=pltpu.CompilerParams(
            dimension_semantics=("arbitrary",),
        ),
    )(emap, z, w1, wg, w2)


def kernel(x, fc1_weight, fc2_weight, indices, counts):
    ind = indices.reshape(-1).astype(jnp.int32)
    counts = counts.astype(jnp.int32)

    tiles = (counts + TILE_M - 1) // TILE_M
    cumtiles = jnp.cumsum(tiles)
    base = (cumtiles - tiles) * TILE_M
    order = jnp.argsort(ind)
    e_sorted = ind[order]
    csum_excl = jnp.cumsum(counts) - counts
    rank = jnp.arange(N, dtype=jnp.int32) - csum_excl[e_sorted]
    dest_sorted = base[e_sorted] + rank
    dest = jnp.zeros((N,), jnp.int32).at[order].set(dest_sorted)
    emap = jnp.minimum(
        jnp.searchsorted(cumtiles, jnp.arange(TMAX, dtype=jnp.int32),
                         side="right"),
        E - 1).astype(jnp.int32)

    z = jnp.zeros((ZROWS, D_MODEL), x.dtype).at[dest_sorted].set(
        x[order // TOP_K])

    w1 = fc1_weight[:, :D_FF, :]
    wg = fc1_weight[:, D_FF:, :]
    zout = _grouped_ffn(z, w1, wg, fc2_weight, emap)

    return zout[dest]
